# initial kernel scaffold (unmeasured)
import jax
import jax.numpy as jnp
from jax import lax
from jax.experimental import pallas as pl
from jax.experimental.pallas import tpu as pltpu

N_DEV = 32
B, SQ, DM = 2, 512, 768
HQ, DH = 8, 64
HD = HQ * DH
SKV_LOC = 512
NCH = N_DEV
CH = SQ // NCH
LW = 640
SCALE = 0.125
BLK = 64


def kernel(x, Wq, K_ext, V_ext, Wo):
    def body(x_ref, wq_ref, k_ref, v_ref, wo_ref, out_ref,
             comb_ref, stage_ref, send_sems, recv_sems):
        me = lax.axis_index("i")
        left = lax.rem(me - 1 + N_DEV, N_DEV)
        right = lax.rem(me + 1, N_DEV)

        bsem = pltpu.get_barrier_semaphore()
        pl.semaphore_signal(bsem, inc=1, device_id=(left,),
                            device_id_type=pl.DeviceIdType.MESH)
        pl.semaphore_signal(bsem, inc=1, device_id=(right,),
                            device_id_type=pl.DeviceIdType.MESH)
        pl.semaphore_wait(bsem, 2)

        qi = lax.broadcasted_iota(jnp.int32, (SQ, SKV_LOC), 0)
        kj = lax.broadcasted_iota(jnp.int32, (SQ, SKV_LOC), 1)
        qb = qi // BLK
        kb = (kj + me * SKV_LOC) // BLK
        mask = (qb == kb) | (kb == 0) | (lax.rem(qb + kb, 3) == 0)

        for b in range(B):
            q2 = jnp.dot(x_ref[b], wq_ref[...],
                         preferred_element_type=jnp.float32)
            l_parts = []
            for h in range(HQ):
                qh = q2[:, h * DH:(h + 1) * DH]
                kh = k_ref[b, :, h * DH:(h + 1) * DH]
                vh = v_ref[b, :, h * DH:(h + 1) * DH]
                s = lax.dot_general(
                    qh, kh, (((1,), (1,)), ((), ())),
                    preferred_element_type=jnp.float32) * SCALE
                w = jnp.where(mask, jnp.exp(s), 0.0)
                comb_ref[b, :, h * DH:(h + 1) * DH] = jnp.dot(
                    w, vh, preferred_element_type=jnp.float32)
                l_parts.append(jnp.sum(w, axis=1, keepdims=True))
            comb_ref[b, :, HD:] = jnp.zeros((SQ, LW - HD), jnp.float32)
            comb_ref[b, :, HD:HD + HQ] = jnp.concatenate(l_parts, axis=1)

        for s in range(N_DEV - 1):
            c_send = lax.rem(me - s + N_DEV, N_DEV)
            c_recv = lax.rem(me - s - 1 + 2 * N_DEV, N_DEV)
            rdma = pltpu.make_async_remote_copy(
                src_ref=comb_ref.at[:, pl.ds(c_send * CH, CH), :],
                dst_ref=stage_ref.at[s],
                send_sem=send_sems.at[s],
                recv_sem=recv_sems.at[s],
                device_id=(right,),
                device_id_type=pl.DeviceIdType.MESH,
            )
            rdma.start()
            rdma.wait()
            cur = comb_ref[:, pl.ds(c_recv * CH, CH), :]
            comb_ref[:, pl.ds(c_recv * CH, CH), :] = cur + stage_ref[s]

        for s in range(N_DEV - 1):
            c_send = lax.rem(me + 1 - s + 2 * N_DEV, N_DEV)
            rdma = pltpu.make_async_remote_copy(
                src_ref=comb_ref.at[:, pl.ds(c_send * CH, CH), :],
                dst_ref=comb_ref.at[:, pl.ds(c_send * CH, CH), :],
                send_sem=send_sems.at[N_DEV - 1 + s],
                recv_sem=recv_sems.at[N_DEV - 1 + s],
                device_id=(right,),
                device_id_type=pl.DeviceIdType.MESH,
            )
            rdma.start()
            rdma.wait()

        for b in range(B):
            lb = comb_ref[b, :, HD:HD + HQ]
            ctx_parts = []
            for h in range(HQ):
                ctx_parts.append(
                    comb_ref[b, :, h * DH:(h + 1) * DH] / lb[:, h:h + 1])
            ctx = jnp.concatenate(ctx_parts, axis=1)
            out_ref[b] = jnp.dot(ctx, wo_ref[...],
                                 preferred_element_type=jnp.float32)

    k2 = K_ext.reshape(B, SKV_LOC, HD)
    v2 = V_ext.reshape(B, SKV_LOC, HD)
    return pl.pallas_call(
        body,
        out_shape=jax.ShapeDtypeStruct((B, SQ, DM), jnp.float32),
        in_specs=[pl.BlockSpec(memory_space=pltpu.VMEM)] * 5,
        out_specs=pl.BlockSpec(memory_space=pltpu.VMEM),
        scratch_shapes=[
            pltpu.VMEM((B, SQ, LW), jnp.float32),
            pltpu.VMEM((N_DEV - 1, B, CH, LW), jnp.float32),
            pltpu.SemaphoreType.DMA((2 * (N_DEV - 1),)),
            pltpu.SemaphoreType.DMA((2 * (N_DEV - 1),)),
        ],
        compiler_params=pltpu.CompilerParams(collective_id=0),
    )(x, Wq, k2, v2, Wo)


# baseline (device time: 69217 ns/iter reference)
import jax
import jax.numpy as jnp
from jax import lax
from jax.experimental import pallas as pl
from jax.experimental.pallas import tpu as pltpu

N_DEV = 32
B, SQ, DM = 2, 512, 768
HQ, DH = 8, 64
HD = HQ * DH
SKV_LOC = 512
CH = SQ // N_DEV
BR = B * CH
LW = 640
SCALE = 0.125
BLK = 64


def _phase1(x2, Wq, K2, V2, Wo):
    def body(x_ref, wq_ref, k_ref, v_ref, wo_ref, och_ref,
             comb_ref, sbuf_ref, rbuf_ref, ssem, rsem):
        me = lax.axis_index("i")

        bsem = pltpu.get_barrier_semaphore()
        for d in range(N_DEV):
            pl.semaphore_signal(bsem, inc=1, device_id=d,
                                device_id_type=pl.DeviceIdType.LOGICAL)
        pl.semaphore_wait(bsem, N_DEV)

        qi = lax.broadcasted_iota(jnp.int32, (SQ, SKV_LOC), 0)
        kj = lax.broadcasted_iota(jnp.int32, (SQ, SKV_LOC), 1)
        qb = qi // BLK
        kb = (kj + me * SKV_LOC) // BLK
        mask = (qb == kb) | (kb == 0) | (lax.rem(qb + kb, 3) == 0)

        for b in range(B):
            rows = slice(b * SQ, (b + 1) * SQ)
            q2 = jnp.dot(x_ref[rows, :], wq_ref[...],
                         preferred_element_type=jnp.float32)
            l_parts = []
            for h in range(HQ):
                qh = q2[:, h * DH:(h + 1) * DH]
                kh = k_ref[rows, h * DH:(h + 1) * DH]
                vh = v_ref[rows, h * DH:(h + 1) * DH]
                s = lax.dot_general(
                    qh, kh, (((1,), (1,)), ((), ())),
                    preferred_element_type=jnp.float32) * SCALE
                w = jnp.where(mask, jnp.exp(s), 0.0)
                comb_ref[rows, h * DH:(h + 1) * DH] = jnp.dot(
                    w, vh, preferred_element_type=jnp.float32)
                l_parts.append(jnp.sum(w, axis=1, keepdims=True))
            comb_ref[rows, HD:] = jnp.zeros((SQ, LW - HD), jnp.float32)
            comb_ref[rows, HD:HD + HQ] = jnp.concatenate(l_parts, axis=1)

        rdmas = []
        for o in range(1, N_DEV):
            d = lax.rem(me + o, N_DEV)
            for b in range(B):
                sbuf_ref[o * BR + b * CH:o * BR + (b + 1) * CH, :] = (
                    comb_ref[pl.ds(b * SQ + d * CH, CH), :]
                    .astype(jnp.bfloat16))
            r = pltpu.make_async_remote_copy(
                src_ref=sbuf_ref.at[pl.ds(o * BR, BR), :],
                dst_ref=rbuf_ref.at[pl.ds(o * BR, BR), :],
                send_sem=ssem.at[o],
                recv_sem=rsem.at[o],
                device_id=d,
                device_id_type=pl.DeviceIdType.LOGICAL,
            )
            r.start()
            rdmas.append(r)
        for r in rdmas:
            r.wait_recv()

        red = jnp.concatenate(
            [comb_ref[pl.ds(b * SQ + me * CH, CH), :] for b in range(B)],
            axis=0)
        for o in range(1, N_DEV):
            red = red + rbuf_ref[o * BR:(o + 1) * BR, :].astype(jnp.float32)

        for b in range(B):
            rb = red[b * CH:(b + 1) * CH, :]
            lb = rb[:, HD:HD + HQ]
            ctx_parts = []
            for h in range(HQ):
                ctx_parts.append(rb[:, h * DH:(h + 1) * DH] / lb[:, h:h + 1])
            ctx = jnp.concatenate(ctx_parts, axis=1)
            och_ref[b * CH:(b + 1) * CH, :] = jnp.dot(
                ctx, wo_ref[...],
                preferred_element_type=jnp.float32).astype(jnp.bfloat16)
        for r in rdmas:
            r.wait_send()

    return pl.pallas_call(
        body,
        out_shape=jax.ShapeDtypeStruct((BR, DM), jnp.bfloat16),
        in_specs=[pl.BlockSpec(memory_space=pltpu.VMEM)] * 5,
        out_specs=pl.BlockSpec(memory_space=pltpu.VMEM),
        scratch_shapes=[
            pltpu.VMEM((B * SQ, LW), jnp.float32),
            pltpu.VMEM((N_DEV * BR, LW), jnp.bfloat16),
            pltpu.VMEM((N_DEV * BR, LW), jnp.bfloat16),
            pltpu.SemaphoreType.DMA((N_DEV,)),
            pltpu.SemaphoreType.DMA((N_DEV,)),
        ],
        compiler_params=pltpu.CompilerParams(collective_id=0),
    )(x2, Wq, K2, V2, Wo)


def _phase2(och):
    def body(och_ref, out_ref, sob_ref, gbuf_ref, ssem, rsem):
        me = lax.axis_index("i")

        bsem = pltpu.get_barrier_semaphore()
        for d in range(N_DEV):
            pl.semaphore_signal(bsem, inc=1, device_id=d,
                                device_id_type=pl.DeviceIdType.LOGICAL)
        pl.semaphore_wait(bsem, N_DEV)

        gbuf_ref[0:BR, :] = och_ref[...]
        for o in range(1, N_DEV):
            sob_ref[o * BR:(o + 1) * BR, :] = och_ref[...]

        rdmas = []
        for o in range(1, N_DEV):
            d = lax.rem(me + o, N_DEV)
            r = pltpu.make_async_remote_copy(
                src_ref=sob_ref.at[pl.ds(o * BR, BR), :],
                dst_ref=gbuf_ref.at[pl.ds(o * BR, BR), :],
                send_sem=ssem.at[o],
                recv_sem=rsem.at[o],
                device_id=d,
                device_id_type=pl.DeviceIdType.LOGICAL,
            )
            r.start()
            rdmas.append(r)
        for r in rdmas:
            r.wait_recv()

        for c in range(N_DEV):
            slot = lax.rem(me - c + N_DEV, N_DEV)
            for b in range(B):
                out_ref[b * SQ + c * CH:b * SQ + (c + 1) * CH, :] = (
                    gbuf_ref[pl.ds(slot * BR + b * CH, CH), :]
                    .astype(jnp.float32))
        for r in rdmas:
            r.wait_send()

    return pl.pallas_call(
        body,
        out_shape=jax.ShapeDtypeStruct((B * SQ, DM), jnp.float32),
        in_specs=[pl.BlockSpec(memory_space=pltpu.VMEM)],
        out_specs=pl.BlockSpec(memory_space=pltpu.VMEM),
        scratch_shapes=[
            pltpu.VMEM((N_DEV * BR, DM), jnp.bfloat16),
            pltpu.VMEM((N_DEV * BR, DM), jnp.bfloat16),
            pltpu.SemaphoreType.DMA((N_DEV,)),
            pltpu.SemaphoreType.DMA((N_DEV,)),
        ],
        compiler_params=pltpu.CompilerParams(collective_id=1),
    )(och)


def kernel(x, Wq, K_ext, V_ext, Wo):
    x2 = x.reshape(B * SQ, DM)
    k2 = K_ext.reshape(B * SKV_LOC, HD)
    v2 = V_ext.reshape(B * SKV_LOC, HD)
    och = _phase1(x2, Wq, k2, v2, Wo)
    out2 = _phase2(och)
    return out2.reshape(B, SQ, DM)
